# Initial kernel scaffold; baseline (speedup 1.0000x reference)
import jax
import jax.numpy as jnp
from jax.experimental import pallas as pl


def kernel(x, table):
    raise NotImplementedError("write your pallas kernel here")


# speedup vs baseline: 3.8057x; 3.8057x over previous
import functools

import jax
import jax.numpy as jnp
from jax import lax
from jax.experimental import pallas as pl
from jax.experimental.pallas import tpu as pltpu
from jax.experimental.pallas import tpu_sc as plsc

NUM_EMBEDDINGS = 100000
OUTPUT_DIM = 64
BATCH = 16384
N_FIELDS = 26

NC = 2
NS = 16
L = 16
NW = NC * NS

TOTAL = BATCH * N_FIELDS
B_PER_W = TOTAL // NW
CHUNK = 128
N_CHUNKS = B_PER_W // CHUNK


def _embed_kernel(x_hbm, table_hbm, out_hbm, idx_v, rows_v, gsem, ssem):
    wid = lax.axis_index("s") * NC + lax.axis_index("c")
    base = wid * B_PER_W

    pltpu.sync_copy(x_hbm.at[pl.ds(base, B_PER_W)], idx_v)

    def shift_body(i, _):
        s = pl.ds(i * L, L)
        idx_v[s] = idx_v[s] + 1
        return ()

    lax.fori_loop(0, B_PER_W // L, shift_body, (), unroll=8)

    def chunk_body(c, _):
        pltpu.async_copy(
            table_hbm.at[idx_v.at[pl.ds(c * CHUNK, CHUNK)]],
            rows_v,
            gsem,
        ).wait()
        pltpu.async_copy(
            rows_v,
            out_hbm.at[pl.ds(base + c * CHUNK, CHUNK), :],
            ssem,
        ).wait()
        return ()

    lax.fori_loop(0, N_CHUNKS, chunk_body, ())


@jax.jit
def kernel(x, table):
    x_flat = x.reshape(TOTAL)
    mesh = plsc.VectorSubcoreMesh(
        core_axis_name="c", subcore_axis_name="s", num_cores=NC, num_subcores=NS
    )
    out = pl.kernel(
        _embed_kernel,
        out_type=jax.ShapeDtypeStruct((TOTAL, OUTPUT_DIM), jnp.float32),
        mesh=mesh,
        scratch_types=[
            pltpu.VMEM((B_PER_W,), jnp.int32),
            pltpu.VMEM((CHUNK, OUTPUT_DIM), jnp.float32),
            pltpu.SemaphoreType.DMA,
            pltpu.SemaphoreType.DMA,
        ],
        compiler_params=pltpu.CompilerParams(use_tc_tiling_on_sc=False),
    )(x_flat, table)
    return out.reshape(BATCH, N_FIELDS, OUTPUT_DIM)


# speedup vs baseline: 4.4453x; 1.1681x over previous
import jax
import jax.numpy as jnp
from jax import lax
from jax.experimental import pallas as pl
from jax.experimental.pallas import tpu as pltpu
from jax.experimental.pallas import tpu_sc as plsc

NUM_EMBEDDINGS = 100000
OUTPUT_DIM = 64
BATCH = 16384
N_FIELDS = 26

NC = 2
NS = 16
L = 16
NW = NC * NS

TOTAL = BATCH * N_FIELDS
B_PER_W = TOTAL // NW
CHUNK = 128
GPS = 4
SUPER = CHUNK * GPS
N_SUPER = B_PER_W // SUPER
N_PAIR = N_SUPER // 2


def _embed_kernel(x_hbm, table_hbm, out_hbm, idx_v, bufs, gsems, ssems):
    wid = lax.axis_index("s") * NC + lax.axis_index("c")
    base = wid * B_PER_W

    pltpu.sync_copy(x_hbm.at[pl.ds(base, B_PER_W)], idx_v)

    def shift_body(i, _):
        s = pl.ds(i * L, L)
        idx_v[s] = idx_v[s] + 1
        return ()

    lax.fori_loop(0, B_PER_W // L, shift_body, (), unroll=8)

    def fire_gathers(s, b):
        for i in range(GPS):
            pltpu.make_async_copy(
                table_hbm.at[idx_v.at[pl.ds(s * SUPER + i * CHUNK, CHUNK)]],
                bufs.at[b, pl.ds(i * CHUNK, CHUNK), :],
                gsems.at[b],
            ).start()

    def wait_gathers(s, b):
        for i in range(GPS):
            pltpu.make_async_copy(
                table_hbm.at[idx_v.at[pl.ds(s * SUPER + i * CHUNK, CHUNK)]],
                bufs.at[b, pl.ds(i * CHUNK, CHUNK), :],
                gsems.at[b],
            ).wait()

    def store(s, b):
        return pltpu.make_async_copy(
            bufs.at[b],
            out_hbm.at[pl.ds(base + s * SUPER, SUPER), :],
            ssems.at[b],
        )

    fire_gathers(0, 0)

    def pair_body(g, _):
        s0 = 2 * g
        s1 = s0 + 1

        @pl.when(g > 0)
        def _():
            store(s1 - 2, 1).wait()

        fire_gathers(s1, 1)
        wait_gathers(s0, 0)
        store(s0, 0).start()
        wait_gathers(s1, 1)

        @pl.when(g + 1 < N_PAIR)
        def _():
            store(s0, 0).wait()
            fire_gathers(s0 + 2, 0)

        store(s1, 1).start()
        return ()

    lax.fori_loop(0, N_PAIR, pair_body, ())

    store(N_SUPER - 2, 0).wait()
    store(N_SUPER - 1, 1).wait()


@jax.jit
def kernel(x, table):
    x_flat = x.reshape(TOTAL)
    mesh = plsc.VectorSubcoreMesh(
        core_axis_name="c", subcore_axis_name="s", num_cores=NC, num_subcores=NS
    )
    out = pl.kernel(
        _embed_kernel,
        out_type=jax.ShapeDtypeStruct((TOTAL, OUTPUT_DIM), jnp.float32),
        mesh=mesh,
        scratch_types=[
            pltpu.VMEM((B_PER_W,), jnp.int32),
            pltpu.VMEM((2, SUPER, OUTPUT_DIM), jnp.float32),
            pltpu.SemaphoreType.DMA((2,)),
            pltpu.SemaphoreType.DMA((2,)),
        ],
        compiler_params=pltpu.CompilerParams(use_tc_tiling_on_sc=False),
    )(x_flat, table)
    return out.reshape(BATCH, N_FIELDS, OUTPUT_DIM)
